# baseline (speedup 1.0000x reference)
import functools

import jax
import jax.numpy as jnp
from jax import lax
from jax.experimental import pallas as pl
from jax.experimental.pallas import tpu as pltpu
from jax.experimental.pallas import tpu_sc as plsc


def _make_deg(N_pad, E, NC, NS, K):
    NW = NC * NS
    epw = E // NW
    n_chunks = epw // K
    rpt = N_pad // NS
    DEPTH = 8
    mesh = plsc.VectorSubcoreMesh(core_axis_name="c", subcore_axis_name="s")

    @functools.partial(
        pl.kernel,
        out_type=jax.ShapeDtypeStruct((NC, N_pad), jnp.float32),
        mesh=mesh,
        scratch_types=[
            pltpu.VMEM((n_chunks, K), jnp.int32),
            pltpu.VMEM((K,), jnp.float32),
            pltpu.VMEM((rpt,), jnp.float32),
            pltpu.VMEM_SHARED((N_pad,), jnp.float32),
            pltpu.SemaphoreType.DMA,
        ],
    )
    def deg_kernel(dst_hbm, out_hbm, didx, ones, zbuf, acc, sem):
        c = lax.axis_index("c")
        s = lax.axis_index("s")
        wid = s * NC + c
        zv = jnp.zeros((16,), jnp.float32)
        ov = jnp.ones((16,), jnp.float32)

        def zfill(i, _):
            zbuf[pl.ds(i * 16, 16)] = zv
            return 0

        lax.fori_loop(0, rpt // 16, zfill, 0)

        def ofill(i, _):
            ones[pl.ds(i * 16, 16)] = ov
            return 0

        lax.fori_loop(0, K // 16, ofill, 0)
        pltpu.sync_copy(dst_hbm.at[wid], didx)
        pltpu.sync_copy(zbuf, acc.at[pl.ds(s * rpt, rpt)])
        plsc.subcore_barrier()

        def chunk(j, _):
            @pl.when(j >= DEPTH)
            def _():
                pltpu.make_async_copy(ones, acc.at[didx.at[0]], sem).wait()

            pltpu.async_copy(ones, acc.at[didx.at[j]], sem, add=True)
            return 0

        lax.fori_loop(0, n_chunks, chunk, 0)
        for _ in range(min(DEPTH, n_chunks)):
            pltpu.make_async_copy(ones, acc.at[didx.at[0]], sem).wait()
        plsc.subcore_barrier()
        pltpu.sync_copy(acc.at[pl.ds(s * rpt, rpt)],
                        out_hbm.at[c, pl.ds(s * rpt, rpt)])

    return deg_kernel


def _make_prop(N, D, E, NC, NS, K, NBUF, tc_tiling):
    NW = NC * NS
    epw = E // NW
    n_chunks = epw // K
    rpt = N // NS
    LA = NBUF - 2
    n_zf = rpt // K
    z_tail = rpt - n_zf * K
    mesh = plsc.VectorSubcoreMesh(core_axis_name="c", subcore_axis_name="s")

    @functools.partial(
        pl.kernel,
        out_type=jax.ShapeDtypeStruct((NC, N, D), jnp.float32),
        mesh=mesh,
        scratch_types=[
            pltpu.VMEM((n_chunks, K), jnp.int32),
            pltpu.VMEM((n_chunks, K), jnp.int32),
            [pltpu.VMEM((K, D), jnp.float32)] * NBUF,
            pltpu.VMEM_SHARED((N, D), jnp.float32),
            [pltpu.SemaphoreType.DMA] * NBUF,
            [pltpu.SemaphoreType.DMA] * NBUF,
        ],
        compiler_params=pltpu.CompilerParams(use_tc_tiling_on_sc=tc_tiling),
    )
    def prop_kernel(table_hbm, src_hbm, dst_hbm, out_hbm, sidx, didx, rows,
                    acc, sg, ss):
        c = lax.axis_index("c")
        s = lax.axis_index("s")
        wid = s * NC + c
        zv = jnp.zeros((16,), jnp.float32)

        def gather_start(j, b):
            pltpu.async_copy(table_hbm.at[sidx.at[j]], rows[b], sg[b])

        def gather_wait(b):
            pltpu.make_async_copy(table_hbm.at[sidx.at[0]], rows[b],
                                  sg[b]).wait()

        def scatter_start(j, b):
            pltpu.async_copy(rows[b], acc.at[didx.at[j]], ss[b], add=True)

        def scatter_wait(b):
            pltpu.make_async_copy(rows[b], acc.at[didx.at[0]], ss[b]).wait()

        def zrow(r, _):
            for j in range(D // 16):
                rows[0][r, pl.ds(j * 16, 16)] = zv
            return 0

        lax.fori_loop(0, K, zrow, 0)

        def zcopy(t, _):
            pltpu.sync_copy(rows[0], acc.at[pl.ds(s * rpt + t * K, K)])
            return 0

        lax.fori_loop(0, n_zf, zcopy, 0)
        if z_tail:
            pltpu.sync_copy(
                rows[0].at[pl.ds(0, z_tail)],
                acc.at[pl.ds(s * rpt + n_zf * K, z_tail)])
        pltpu.sync_copy(src_hbm.at[wid], sidx)
        pltpu.sync_copy(dst_hbm.at[wid], didx)
        for j in range(LA):
            gather_start(j, j)
        plsc.subcore_barrier()

        def step(j, b):
            @pl.when(j >= NBUF - LA)
            def _():
                scatter_wait((b + LA) % NBUF)

            @pl.when(j + LA < n_chunks)
            def _():
                gather_start(j + LA, (b + LA) % NBUF)

            gather_wait(b)
            scatter_start(j, b)

        n_main = (n_chunks // NBUF) * NBUF

        def body(j4, _):
            for b in range(NBUF):
                step(j4 * NBUF + b, b)
            return 0

        lax.fori_loop(0, n_main // NBUF, body, 0)
        for j in range(n_main, n_chunks):
            step(j, j % NBUF)
        for j in range(max(0, n_chunks - (NBUF - LA)), n_chunks):
            scatter_wait(j % NBUF)
        plsc.subcore_barrier()
        if tc_tiling:
            bounds = [(sv * N // NS) // 8 * 8 for sv in range(NS)] + [N]
            for sv in range(NS):
                st, ln = bounds[sv], bounds[sv + 1] - bounds[sv]

                @pl.when(s == sv)
                def _(st=st, ln=ln):
                    pltpu.sync_copy(acc.at[pl.ds(st, ln)],
                                    out_hbm.at[c, pl.ds(st, ln)])
        else:
            pltpu.sync_copy(acc.at[pl.ds(s * rpt, rpt)],
                            out_hbm.at[c, pl.ds(s * rpt, rpt)])

    return prop_kernel


def _scale_body(p_ref, x_ref, dinv_ref, xs_ref, *, NC):
    deg = p_ref[0]
    for c in range(1, NC):
        deg = deg + p_ref[c]
    dinv = lax.rsqrt(deg + 1.0)
    dinv_ref[...] = dinv
    xs_ref[...] = x_ref[...] * dinv


def _mlp_body(p_ref, xs_ref, dinv_ref, W1_ref, b1_ref, W2_ref, ts_ref, *, NC):
    agg = p_ref[0]
    for c in range(1, NC):
        agg = agg + p_ref[c]
    ax = (agg + xs_ref[...]) * dinv_ref[...]
    h = jnp.dot(ax, W1_ref[...],
                preferred_element_type=jnp.float32) + b1_ref[...]
    h = jnp.maximum(h, 0.0)
    t = jnp.dot(h, W2_ref[...],
                preferred_element_type=jnp.float32)
    ts_ref[...] = t * dinv_ref[...]


def _out_body(p_ref, ts_ref, dinv_ref, b2_ref, o_ref, *, NC, D_OUT):
    agg = p_ref[0]
    for c in range(1, NC):
        agg = agg + p_ref[c]
    o = (agg + ts_ref[...]) * dinv_ref[...] + b2_ref[...]
    o = o[:, :D_OUT]
    m = jnp.max(o, axis=1, keepdims=True)
    e = jnp.exp(o - m)
    ssum = jnp.sum(e, axis=1, keepdims=True)
    o_ref[...] = o - m - jnp.log(ssum)


def kernel(x, edge_index, W1, b1, W2, b2):
    N, D_IN = x.shape
    E = edge_index.shape[1]
    D_HID = W1.shape[1]
    D_OUT = W2.shape[1]
    D_OUT_P = 48

    src = edge_index[0].astype(jnp.int32)
    dst = edge_index[1].astype(jnp.int32)
    W2p = jnp.pad(W2, ((0, 0), (0, D_OUT_P - D_OUT)))
    b1r = b1.reshape(1, D_HID)
    b2p = jnp.pad(b2, (0, D_OUT_P - D_OUT)).reshape(1, D_OUT_P)

    info = plsc.get_sparse_core_info()
    NC, NS = info.num_cores, info.num_subcores
    NW = NC * NS
    K = 80
    epw = E // NW
    N_pad = ((N + 16 * NS - 1) // (16 * NS)) * (16 * NS)

    src3 = src.reshape(NW, epw // K, K)
    dst3 = dst.reshape(NW, epw // K, K)

    deg_raw = _make_deg(N_pad, E, NC, NS, K)(dst3)
    deg_p = deg_raw[:, :N].reshape(NC, N, 1)

    BR = 2000
    dinv, xs = pl.pallas_call(
        functools.partial(_scale_body, NC=NC),
        grid=(N // BR,),
        in_specs=[
            pl.BlockSpec((NC, BR, 1), lambda i: (0, i, 0)),
            pl.BlockSpec((BR, D_IN), lambda i: (i, 0)),
        ],
        out_specs=[
            pl.BlockSpec((BR, 1), lambda i: (i, 0)),
            pl.BlockSpec((BR, D_IN), lambda i: (i, 0)),
        ],
        out_shape=[
            jax.ShapeDtypeStruct((N, 1), jnp.float32),
            jax.ShapeDtypeStruct((N, D_IN), jnp.float32),
        ],
    )(deg_p, x)

    p1 = _make_prop(N, D_IN, E, NC, NS, K, 3, False)(xs, src3, dst3)

    ts = pl.pallas_call(
        functools.partial(_mlp_body, NC=NC),
        grid=(N // BR,),
        in_specs=[
            pl.BlockSpec((NC, BR, D_IN), lambda i: (0, i, 0)),
            pl.BlockSpec((BR, D_IN), lambda i: (i, 0)),
            pl.BlockSpec((BR, 1), lambda i: (i, 0)),
            pl.BlockSpec((D_IN, D_HID), lambda i: (0, 0)),
            pl.BlockSpec((1, D_HID), lambda i: (0, 0)),
            pl.BlockSpec((D_HID, D_OUT_P), lambda i: (0, 0)),
        ],
        out_specs=pl.BlockSpec((BR, D_OUT_P), lambda i: (i, 0)),
        out_shape=jax.ShapeDtypeStruct((N, D_OUT_P), jnp.float32),
    )(p1, xs, dinv, W1, b1r, W2p)

    p2 = _make_prop(N, D_OUT_P, E, NC, NS, K, 5, False)(ts, src3, dst3)

    out = pl.pallas_call(
        functools.partial(_out_body, NC=NC, D_OUT=D_OUT),
        grid=(N // BR,),
        in_specs=[
            pl.BlockSpec((NC, BR, D_OUT_P), lambda i: (0, i, 0)),
            pl.BlockSpec((BR, D_OUT_P), lambda i: (i, 0)),
            pl.BlockSpec((BR, 1), lambda i: (i, 0)),
            pl.BlockSpec((1, D_OUT_P), lambda i: (0, 0)),
        ],
        out_specs=pl.BlockSpec((BR, D_OUT), lambda i: (i, 0)),
        out_shape=jax.ShapeDtypeStruct((N, D_OUT), jnp.float32),
    )(p2, ts, dinv, b2p)
    return out


# speedup vs baseline: 1.0005x; 1.0005x over previous
import functools

import jax
import jax.numpy as jnp
from jax import lax
from jax.experimental import pallas as pl
from jax.experimental.pallas import tpu as pltpu
from jax.experimental.pallas import tpu_sc as plsc


def _make_deg(N_pad, E, NC, NS, K):
    NW = NC * NS
    epw = E // NW
    n_chunks = epw // K
    rpt = N_pad // NS
    DEPTH = 8
    mesh = plsc.VectorSubcoreMesh(core_axis_name="c", subcore_axis_name="s")

    @functools.partial(
        pl.kernel,
        out_type=jax.ShapeDtypeStruct((NC, N_pad), jnp.float32),
        mesh=mesh,
        scratch_types=[
            pltpu.VMEM((n_chunks, K), jnp.int32),
            pltpu.VMEM((K,), jnp.float32),
            pltpu.VMEM((rpt,), jnp.float32),
            pltpu.VMEM_SHARED((N_pad,), jnp.float32),
            pltpu.SemaphoreType.DMA,
        ],
    )
    def deg_kernel(dst_hbm, out_hbm, didx, ones, zbuf, acc, sem):
        c = lax.axis_index("c")
        s = lax.axis_index("s")
        wid = s * NC + c
        zv = jnp.zeros((16,), jnp.float32)
        ov = jnp.ones((16,), jnp.float32)

        def zfill(i, _):
            zbuf[pl.ds(i * 16, 16)] = zv
            return 0

        lax.fori_loop(0, rpt // 16, zfill, 0)

        def ofill(i, _):
            ones[pl.ds(i * 16, 16)] = ov
            return 0

        lax.fori_loop(0, K // 16, ofill, 0)
        pltpu.sync_copy(dst_hbm.at[wid], didx)
        pltpu.sync_copy(zbuf, acc.at[pl.ds(s * rpt, rpt)])
        plsc.subcore_barrier()

        def chunk(j, _):
            @pl.when(j >= DEPTH)
            def _():
                pltpu.make_async_copy(ones, acc.at[didx.at[0]], sem).wait()

            pltpu.async_copy(ones, acc.at[didx.at[j]], sem, add=True)
            return 0

        lax.fori_loop(0, n_chunks, chunk, 0)
        for _ in range(min(DEPTH, n_chunks)):
            pltpu.make_async_copy(ones, acc.at[didx.at[0]], sem).wait()
        plsc.subcore_barrier()
        pltpu.sync_copy(acc.at[pl.ds(s * rpt, rpt)],
                        out_hbm.at[c, pl.ds(s * rpt, rpt)])

    return deg_kernel


def _make_prop(N, D, E, NC, NS, K, NBUF):
    NW = NC * NS
    epw = E // NW
    n_chunks = epw // K
    rpt = N // NS
    LA = NBUF - 2
    n_zf = rpt // K
    z_tail = rpt - n_zf * K
    mesh = plsc.VectorSubcoreMesh(core_axis_name="c", subcore_axis_name="s")

    @functools.partial(
        pl.kernel,
        out_type=jax.ShapeDtypeStruct((NC, N, D), jnp.float32),
        mesh=mesh,
        scratch_types=[
            pltpu.VMEM((n_chunks, K), jnp.int32),
            pltpu.VMEM((n_chunks, K), jnp.int32),
            [pltpu.VMEM((K, D), jnp.float32)] * NBUF,
            pltpu.VMEM_SHARED((N, D), jnp.float32),
            [pltpu.SemaphoreType.DMA] * NBUF,
            [pltpu.SemaphoreType.DMA] * NBUF,
        ],
        compiler_params=pltpu.CompilerParams(use_tc_tiling_on_sc=False),
    )
    def prop_kernel(table_hbm, src_hbm, dst_hbm, out_hbm, sidx, didx, rows,
                    acc, sg, ss):
        c = lax.axis_index("c")
        s = lax.axis_index("s")
        wid = s * NC + c
        zv = jnp.zeros((16,), jnp.float32)

        def gather_start(j, b):
            pltpu.async_copy(table_hbm.at[sidx.at[j]], rows[b], sg[b])

        def gather_wait(b):
            pltpu.make_async_copy(table_hbm.at[sidx.at[0]], rows[b],
                                  sg[b]).wait()

        def scatter_start(j, b):
            pltpu.async_copy(rows[b], acc.at[didx.at[j]], ss[b], add=True)

        def scatter_wait(b):
            pltpu.make_async_copy(rows[b], acc.at[didx.at[0]], ss[b]).wait()

        def zrow(r, _):
            for j in range(D // 16):
                rows[0][r, pl.ds(j * 16, 16)] = zv
            return 0

        lax.fori_loop(0, K, zrow, 0)

        def zcopy(t, _):
            pltpu.sync_copy(rows[0], acc.at[pl.ds(s * rpt + t * K, K)])
            return 0

        lax.fori_loop(0, n_zf, zcopy, 0)
        if z_tail:
            pltpu.sync_copy(
                rows[0].at[pl.ds(0, z_tail)],
                acc.at[pl.ds(s * rpt + n_zf * K, z_tail)])
        pltpu.sync_copy(src_hbm.at[wid], sidx)
        pltpu.sync_copy(dst_hbm.at[wid], didx)
        for j in range(LA):
            gather_start(j, j)
        plsc.subcore_barrier()

        def step(j, b):
            @pl.when(j >= NBUF - LA)
            def _():
                scatter_wait((b + LA) % NBUF)

            @pl.when(j + LA < n_chunks)
            def _():
                gather_start(j + LA, (b + LA) % NBUF)

            gather_wait(b)
            scatter_start(j, b)

        n_main = (n_chunks // NBUF) * NBUF

        def body(j4, _):
            for b in range(NBUF):
                step(j4 * NBUF + b, b)
            return 0

        lax.fori_loop(0, n_main // NBUF, body, 0)
        for j in range(n_main, n_chunks):
            step(j, j % NBUF)
        for j in range(max(0, n_chunks - (NBUF - LA)), n_chunks):
            scatter_wait(j % NBUF)
        plsc.subcore_barrier()
        pltpu.sync_copy(acc.at[pl.ds(s * rpt, rpt)],
                        out_hbm.at[c, pl.ds(s * rpt, rpt)])

    return prop_kernel


def _scale_body(p_ref, x_ref, dinv_ref, xs_ref, *, NC):
    deg = p_ref[0]
    for c in range(1, NC):
        deg = deg + p_ref[c]
    dinv = lax.rsqrt(deg + 1.0)
    dinv_ref[...] = dinv
    xs_ref[...] = x_ref[...] * dinv


def _mlp_body(p_ref, xs_ref, dinv_ref, W1_ref, b1_ref, W2_ref, ts_ref, *, NC):
    agg = p_ref[0]
    for c in range(1, NC):
        agg = agg + p_ref[c]
    ax = (agg + xs_ref[...]) * dinv_ref[...]
    h = jnp.dot(ax, W1_ref[...],
                preferred_element_type=jnp.float32) + b1_ref[...]
    h = jnp.maximum(h, 0.0)
    t = jnp.dot(h, W2_ref[...],
                preferred_element_type=jnp.float32)
    ts_ref[...] = t * dinv_ref[...]


def _out_body(p_ref, ts_ref, dinv_ref, b2_ref, o_ref, *, NC, D_OUT):
    agg = p_ref[0]
    for c in range(1, NC):
        agg = agg + p_ref[c]
    o = (agg + ts_ref[...]) * dinv_ref[...] + b2_ref[...]
    o = o[:, :D_OUT]
    m = jnp.max(o, axis=1, keepdims=True)
    e = jnp.exp(o - m)
    ssum = jnp.sum(e, axis=1, keepdims=True)
    o_ref[...] = o - m - jnp.log(ssum)


def kernel(x, edge_index, W1, b1, W2, b2):
    N, D_IN = x.shape
    E = edge_index.shape[1]
    D_HID = W1.shape[1]
    D_OUT = W2.shape[1]
    D_OUT_P = 48

    src = edge_index[0].astype(jnp.int32)
    dst = edge_index[1].astype(jnp.int32)
    W2p = jnp.pad(W2, ((0, 0), (0, D_OUT_P - D_OUT)))
    b1r = b1.reshape(1, D_HID)
    b2p = jnp.pad(b2, (0, D_OUT_P - D_OUT)).reshape(1, D_OUT_P)

    info = plsc.get_sparse_core_info()
    NC, NS = info.num_cores, info.num_subcores
    NW = NC * NS
    K = 80
    epw = E // NW
    N_pad = ((N + 16 * NS - 1) // (16 * NS)) * (16 * NS)

    src3 = src.reshape(NW, epw // K, K)
    dst3 = dst.reshape(NW, epw // K, K)

    deg_raw = _make_deg(N_pad, E, NC, NS, K)(dst3)
    deg_p = deg_raw[:, :N].reshape(NC, N, 1)

    BR = 2000
    dinv, xs = pl.pallas_call(
        functools.partial(_scale_body, NC=NC),
        grid=(N // BR,),
        in_specs=[
            pl.BlockSpec((NC, BR, 1), lambda i: (0, i, 0)),
            pl.BlockSpec((BR, D_IN), lambda i: (i, 0)),
        ],
        out_specs=[
            pl.BlockSpec((BR, 1), lambda i: (i, 0)),
            pl.BlockSpec((BR, D_IN), lambda i: (i, 0)),
        ],
        out_shape=[
            jax.ShapeDtypeStruct((N, 1), jnp.float32),
            jax.ShapeDtypeStruct((N, D_IN), jnp.float32),
        ],
    )(deg_p, x)

    p1 = _make_prop(N, D_IN, E, NC, NS, K, 3)(xs, src3, dst3)

    ts = pl.pallas_call(
        functools.partial(_mlp_body, NC=NC),
        grid=(N // BR,),
        in_specs=[
            pl.BlockSpec((NC, BR, D_IN), lambda i: (0, i, 0)),
            pl.BlockSpec((BR, D_IN), lambda i: (i, 0)),
            pl.BlockSpec((BR, 1), lambda i: (i, 0)),
            pl.BlockSpec((D_IN, D_HID), lambda i: (0, 0)),
            pl.BlockSpec((1, D_HID), lambda i: (0, 0)),
            pl.BlockSpec((D_HID, D_OUT_P), lambda i: (0, 0)),
        ],
        out_specs=pl.BlockSpec((BR, D_OUT_P), lambda i: (i, 0)),
        out_shape=jax.ShapeDtypeStruct((N, D_OUT_P), jnp.float32),
    )(p1, xs, dinv, W1, b1r, W2p)

    p2 = _make_prop(N, D_OUT_P, E, NC, NS, K, 5)(ts, src3, dst3)

    out = pl.pallas_call(
        functools.partial(_out_body, NC=NC, D_OUT=D_OUT),
        grid=(N // BR,),
        in_specs=[
            pl.BlockSpec((NC, BR, D_OUT_P), lambda i: (0, i, 0)),
            pl.BlockSpec((BR, D_OUT_P), lambda i: (i, 0)),
            pl.BlockSpec((BR, 1), lambda i: (i, 0)),
            pl.BlockSpec((1, D_OUT_P), lambda i: (0, 0)),
        ],
        out_specs=pl.BlockSpec((BR, D_OUT), lambda i: (i, 0)),
        out_shape=jax.ShapeDtypeStruct((N, D_OUT), jnp.float32),
    )(p2, ts, dinv, b2p)
    return out


# speedup vs baseline: 1.0201x; 1.0196x over previous
import functools

import jax
import jax.numpy as jnp
from jax import lax
from jax.experimental import pallas as pl
from jax.experimental.pallas import tpu as pltpu
from jax.experimental.pallas import tpu_sc as plsc


def _make_deg(N_pad, E, NC, NS, K):
    NW = NC * NS
    epw = E // NW
    n_chunks = epw // K
    rpt = N_pad // NS
    DEPTH = 8
    mesh = plsc.VectorSubcoreMesh(core_axis_name="c", subcore_axis_name="s")

    @functools.partial(
        pl.kernel,
        out_type=jax.ShapeDtypeStruct((NC, N_pad), jnp.float32),
        mesh=mesh,
        scratch_types=[
            pltpu.VMEM((n_chunks, K), jnp.int32),
            pltpu.VMEM((K,), jnp.float32),
            pltpu.VMEM((rpt,), jnp.float32),
            pltpu.VMEM_SHARED((N_pad,), jnp.float32),
            pltpu.SemaphoreType.DMA,
        ],
    )
    def deg_kernel(dst_hbm, out_hbm, didx, ones, zbuf, acc, sem):
        c = lax.axis_index("c")
        s = lax.axis_index("s")
        wid = s * NC + c
        zv = jnp.zeros((16,), jnp.float32)
        ov = jnp.ones((16,), jnp.float32)

        def zfill(i, _):
            zbuf[pl.ds(i * 16, 16)] = zv
            return 0

        lax.fori_loop(0, rpt // 16, zfill, 0)

        def ofill(i, _):
            ones[pl.ds(i * 16, 16)] = ov
            return 0

        lax.fori_loop(0, K // 16, ofill, 0)
        pltpu.sync_copy(dst_hbm.at[wid], didx)
        pltpu.sync_copy(zbuf, acc.at[pl.ds(s * rpt, rpt)])
        plsc.subcore_barrier()

        def chunk(j, _):
            @pl.when(j >= DEPTH)
            def _():
                pltpu.make_async_copy(ones, acc.at[didx.at[0]], sem).wait()

            pltpu.async_copy(ones, acc.at[didx.at[j]], sem, add=True)
            return 0

        lax.fori_loop(0, n_chunks, chunk, 0)
        for _ in range(min(DEPTH, n_chunks)):
            pltpu.make_async_copy(ones, acc.at[didx.at[0]], sem).wait()
        plsc.subcore_barrier()
        pltpu.sync_copy(acc.at[pl.ds(s * rpt, rpt)],
                        out_hbm.at[c, pl.ds(s * rpt, rpt)])

    return deg_kernel


def _make_prop(N, D, E, NC, NS, K, NBUF):
    NW = NC * NS
    epw = E // NW
    n_chunks = epw // K
    rpt = N // NS
    LA = NBUF - 2
    n_zf = rpt // K
    z_tail = rpt - n_zf * K
    mesh = plsc.VectorSubcoreMesh(core_axis_name="c", subcore_axis_name="s")

    @functools.partial(
        pl.kernel,
        out_type=jax.ShapeDtypeStruct((NC, N, D), jnp.float32),
        mesh=mesh,
        scratch_types=[
            pltpu.VMEM((n_chunks, K), jnp.int32),
            pltpu.VMEM((n_chunks, K), jnp.int32),
            [pltpu.VMEM((K, D), jnp.float32)] * NBUF,
            pltpu.VMEM_SHARED((N, D), jnp.float32),
            [pltpu.SemaphoreType.DMA] * NBUF,
            [pltpu.SemaphoreType.DMA] * NBUF,
        ],
        compiler_params=pltpu.CompilerParams(use_tc_tiling_on_sc=False),
    )
    def prop_kernel(table_hbm, src_hbm, dst_hbm, out_hbm, sidx, didx, rows,
                    acc, sg, ss):
        c = lax.axis_index("c")
        s = lax.axis_index("s")
        wid = s * NC + c
        zv = jnp.zeros((16,), jnp.float32)

        def gather_start(j, b):
            pltpu.async_copy(table_hbm.at[sidx.at[j]], rows[b], sg[b])

        def gather_wait(b):
            pltpu.make_async_copy(table_hbm.at[sidx.at[0]], rows[b],
                                  sg[b]).wait()

        def scatter_start(j, b):
            pltpu.async_copy(rows[b], acc.at[didx.at[j]], ss[b], add=True)

        def scatter_wait(b):
            pltpu.make_async_copy(rows[b], acc.at[didx.at[0]], ss[b]).wait()

        def zrow(r, _):
            for j in range(D // 16):
                rows[0][r, pl.ds(j * 16, 16)] = zv
            return 0

        lax.fori_loop(0, K, zrow, 0)

        def zcopy(t, _):
            pltpu.sync_copy(rows[0], acc.at[pl.ds(s * rpt + t * K, K)])
            return 0

        lax.fori_loop(0, n_zf, zcopy, 0)
        if z_tail:
            pltpu.sync_copy(
                rows[0].at[pl.ds(0, z_tail)],
                acc.at[pl.ds(s * rpt + n_zf * K, z_tail)])
        pltpu.sync_copy(src_hbm.at[wid], sidx)
        pltpu.sync_copy(dst_hbm.at[wid], didx)
        for j in range(LA):
            gather_start(j, j)
        plsc.subcore_barrier()

        def step(j, b):
            @pl.when(j >= NBUF - LA)
            def _():
                scatter_wait((b + LA) % NBUF)

            @pl.when(j + LA < n_chunks)
            def _():
                gather_start(j + LA, (b + LA) % NBUF)

            gather_wait(b)
            scatter_start(j, b)

        n_main = (n_chunks // NBUF) * NBUF

        def body(j4, _):
            for b in range(NBUF):
                step(j4 * NBUF + b, b)
            return 0

        lax.fori_loop(0, n_main // NBUF, body, 0)
        for j in range(n_main, n_chunks):
            step(j, j % NBUF)
        for j in range(max(0, n_chunks - (NBUF - LA)), n_chunks):
            scatter_wait(j % NBUF)
        plsc.subcore_barrier()
        pltpu.sync_copy(acc.at[pl.ds(s * rpt, rpt)],
                        out_hbm.at[c, pl.ds(s * rpt, rpt)])

    return prop_kernel


def _scale_body(p_ref, x_ref, dinv_ref, xs_ref, *, NC):
    deg = p_ref[0]
    for c in range(1, NC):
        deg = deg + p_ref[c]
    dinv = lax.rsqrt(deg + 1.0)
    dinv_ref[...] = dinv
    xs_ref[...] = x_ref[...] * dinv


def _mlp_body(p_ref, xs_ref, dinv_ref, W1_ref, b1_ref, W2_ref, ts_ref, *, NC):
    agg = p_ref[0]
    for c in range(1, NC):
        agg = agg + p_ref[c]
    ax = (agg + xs_ref[...]) * dinv_ref[...]
    h = jnp.dot(ax, W1_ref[...],
                preferred_element_type=jnp.float32) + b1_ref[...]
    h = jnp.maximum(h, 0.0)
    t = jnp.dot(h, W2_ref[...],
                preferred_element_type=jnp.float32)
    ts_ref[...] = t * dinv_ref[...]


def _out_body(p_ref, ts_ref, dinv_ref, b2_ref, o_ref, *, NC, D_OUT):
    agg = p_ref[0]
    for c in range(1, NC):
        agg = agg + p_ref[c]
    o = (agg + ts_ref[...]) * dinv_ref[...] + b2_ref[...]
    o = o[:, :D_OUT]
    m = jnp.max(o, axis=1, keepdims=True)
    e = jnp.exp(o - m)
    ssum = jnp.sum(e, axis=1, keepdims=True)
    o_ref[...] = o - m - jnp.log(ssum)


def kernel(x, edge_index, W1, b1, W2, b2):
    N, D_IN = x.shape
    E = edge_index.shape[1]
    D_HID = W1.shape[1]
    D_OUT = W2.shape[1]
    D_OUT_P = 48

    src = edge_index[0].astype(jnp.int32)
    dst = edge_index[1].astype(jnp.int32)
    W2p = jnp.pad(W2, ((0, 0), (0, D_OUT_P - D_OUT)))
    b1r = b1.reshape(1, D_HID)
    b2p = jnp.pad(b2, (0, D_OUT_P - D_OUT)).reshape(1, D_OUT_P)

    info = plsc.get_sparse_core_info()
    NC, NS = info.num_cores, info.num_subcores
    NW = NC * NS
    K = 80
    epw = E // NW
    N_pad = ((N + 16 * NS - 1) // (16 * NS)) * (16 * NS)

    src3 = src.reshape(NW, epw // K, K)
    dst3 = dst.reshape(NW, epw // K, K)

    deg_raw = _make_deg(N_pad, E, NC, NS, K)(dst3)
    deg_p = deg_raw[:, :N].reshape(NC, N, 1)

    BR = 5000
    dinv, xs = pl.pallas_call(
        functools.partial(_scale_body, NC=NC),
        grid=(N // BR,),
        in_specs=[
            pl.BlockSpec((NC, BR, 1), lambda i: (0, i, 0)),
            pl.BlockSpec((BR, D_IN), lambda i: (i, 0)),
        ],
        out_specs=[
            pl.BlockSpec((BR, 1), lambda i: (i, 0)),
            pl.BlockSpec((BR, D_IN), lambda i: (i, 0)),
        ],
        out_shape=[
            jax.ShapeDtypeStruct((N, 1), jnp.float32),
            jax.ShapeDtypeStruct((N, D_IN), jnp.float32),
        ],
    )(deg_p, x)

    p1 = _make_prop(N, D_IN, E, NC, NS, K, 3)(xs, src3, dst3)

    ts = pl.pallas_call(
        functools.partial(_mlp_body, NC=NC),
        grid=(N // BR,),
        in_specs=[
            pl.BlockSpec((NC, BR, D_IN), lambda i: (0, i, 0)),
            pl.BlockSpec((BR, D_IN), lambda i: (i, 0)),
            pl.BlockSpec((BR, 1), lambda i: (i, 0)),
            pl.BlockSpec((D_IN, D_HID), lambda i: (0, 0)),
            pl.BlockSpec((1, D_HID), lambda i: (0, 0)),
            pl.BlockSpec((D_HID, D_OUT_P), lambda i: (0, 0)),
        ],
        out_specs=pl.BlockSpec((BR, D_OUT_P), lambda i: (i, 0)),
        out_shape=jax.ShapeDtypeStruct((N, D_OUT_P), jnp.float32),
    )(p1, xs, dinv, W1, b1r, W2p)

    p2 = _make_prop(N, D_OUT_P, E, NC, NS, K, 5)(ts, src3, dst3)

    out = pl.pallas_call(
        functools.partial(_out_body, NC=NC, D_OUT=D_OUT),
        grid=(N // BR,),
        in_specs=[
            pl.BlockSpec((NC, BR, D_OUT_P), lambda i: (0, i, 0)),
            pl.BlockSpec((BR, D_OUT_P), lambda i: (i, 0)),
            pl.BlockSpec((BR, 1), lambda i: (i, 0)),
            pl.BlockSpec((1, D_OUT_P), lambda i: (0, 0)),
        ],
        out_specs=pl.BlockSpec((BR, D_OUT), lambda i: (i, 0)),
        out_shape=jax.ShapeDtypeStruct((N, D_OUT), jnp.float32),
    )(p2, ts, dinv, b2p)
    return out


# speedup vs baseline: 1.0327x; 1.0124x over previous
import functools

import jax
import jax.numpy as jnp
from jax import lax
from jax.experimental import pallas as pl
from jax.experimental.pallas import tpu as pltpu
from jax.experimental.pallas import tpu_sc as plsc


def _make_deg(N_pad, E, NC, NS, K):
    NW = NC * NS
    epw = E // NW
    n_chunks = epw // K
    rpt = N_pad // NS
    DEPTH = 16
    mesh = plsc.VectorSubcoreMesh(core_axis_name="c", subcore_axis_name="s")

    @functools.partial(
        pl.kernel,
        out_type=jax.ShapeDtypeStruct((NC, N_pad), jnp.float32),
        mesh=mesh,
        scratch_types=[
            pltpu.VMEM((n_chunks, K), jnp.int32),
            pltpu.VMEM((K,), jnp.float32),
            pltpu.VMEM((rpt,), jnp.float32),
            pltpu.VMEM_SHARED((N_pad,), jnp.float32),
            pltpu.SemaphoreType.DMA,
        ],
    )
    def deg_kernel(dst_hbm, out_hbm, didx, ones, zbuf, acc, sem):
        c = lax.axis_index("c")
        s = lax.axis_index("s")
        wid = s * NC + c
        zv = jnp.zeros((16,), jnp.float32)
        ov = jnp.ones((16,), jnp.float32)

        def zfill(i, _):
            zbuf[pl.ds(i * 16, 16)] = zv
            return 0

        lax.fori_loop(0, rpt // 16, zfill, 0)

        def ofill(i, _):
            ones[pl.ds(i * 16, 16)] = ov
            return 0

        lax.fori_loop(0, K // 16, ofill, 0)
        pltpu.sync_copy(dst_hbm.at[wid], didx)
        pltpu.sync_copy(zbuf, acc.at[pl.ds(s * rpt, rpt)])
        plsc.subcore_barrier()

        def chunk(j, _):
            @pl.when(j >= DEPTH)
            def _():
                pltpu.make_async_copy(ones, acc.at[didx.at[0]], sem).wait()

            pltpu.async_copy(ones, acc.at[didx.at[j]], sem, add=True)
            return 0

        lax.fori_loop(0, n_chunks, chunk, 0)
        for _ in range(min(DEPTH, n_chunks)):
            pltpu.make_async_copy(ones, acc.at[didx.at[0]], sem).wait()
        plsc.subcore_barrier()
        pltpu.sync_copy(acc.at[pl.ds(s * rpt, rpt)],
                        out_hbm.at[c, pl.ds(s * rpt, rpt)])

    return deg_kernel


def _make_prop(N, D, E, NC, NS, K, NBUF):
    NW = NC * NS
    epw = E // NW
    n_chunks = epw // K
    rpt = N // NS
    LA = NBUF - 2
    n_zf = rpt // K
    z_tail = rpt - n_zf * K
    mesh = plsc.VectorSubcoreMesh(core_axis_name="c", subcore_axis_name="s")

    @functools.partial(
        pl.kernel,
        out_type=jax.ShapeDtypeStruct((NC, N, D), jnp.float32),
        mesh=mesh,
        scratch_types=[
            pltpu.VMEM((n_chunks, K), jnp.int32),
            pltpu.VMEM((n_chunks, K), jnp.int32),
            [pltpu.VMEM((K, D), jnp.float32)] * NBUF,
            pltpu.VMEM_SHARED((N, D), jnp.float32),
            [pltpu.SemaphoreType.DMA] * NBUF,
            [pltpu.SemaphoreType.DMA] * NBUF,
        ],
        compiler_params=pltpu.CompilerParams(use_tc_tiling_on_sc=False),
    )
    def prop_kernel(table_hbm, src_hbm, dst_hbm, out_hbm, sidx, didx, rows,
                    acc, sg, ss):
        c = lax.axis_index("c")
        s = lax.axis_index("s")
        wid = s * NC + c
        zv = jnp.zeros((16,), jnp.float32)

        def gather_start(j, b):
            pltpu.async_copy(table_hbm.at[sidx.at[j]], rows[b], sg[b])

        def gather_wait(b):
            pltpu.make_async_copy(table_hbm.at[sidx.at[0]], rows[b],
                                  sg[b]).wait()

        def scatter_start(j, b):
            pltpu.async_copy(rows[b], acc.at[didx.at[j]], ss[b], add=True)

        def scatter_wait(b):
            pltpu.make_async_copy(rows[b], acc.at[didx.at[0]], ss[b]).wait()

        def zrow(r, _):
            for j in range(D // 16):
                rows[0][r, pl.ds(j * 16, 16)] = zv
            return 0

        lax.fori_loop(0, K, zrow, 0)

        def zcopy(t, _):
            pltpu.sync_copy(rows[0], acc.at[pl.ds(s * rpt + t * K, K)])
            return 0

        lax.fori_loop(0, n_zf, zcopy, 0)
        if z_tail:
            pltpu.sync_copy(
                rows[0].at[pl.ds(0, z_tail)],
                acc.at[pl.ds(s * rpt + n_zf * K, z_tail)])
        pltpu.sync_copy(src_hbm.at[wid], sidx)
        pltpu.sync_copy(dst_hbm.at[wid], didx)
        for j in range(LA):
            gather_start(j, j)
        plsc.subcore_barrier()

        def step(j, b):
            @pl.when(j >= NBUF - LA)
            def _():
                scatter_wait((b + LA) % NBUF)

            @pl.when(j + LA < n_chunks)
            def _():
                gather_start(j + LA, (b + LA) % NBUF)

            gather_wait(b)
            scatter_start(j, b)

        n_main = (n_chunks // NBUF) * NBUF

        def body(j4, _):
            for b in range(NBUF):
                step(j4 * NBUF + b, b)
            return 0

        lax.fori_loop(0, n_main // NBUF, body, 0)
        for j in range(n_main, n_chunks):
            step(j, j % NBUF)
        for j in range(max(0, n_chunks - (NBUF - LA)), n_chunks):
            scatter_wait(j % NBUF)
        plsc.subcore_barrier()
        pltpu.sync_copy(acc.at[pl.ds(s * rpt, rpt)],
                        out_hbm.at[c, pl.ds(s * rpt, rpt)])

    return prop_kernel


def _scale_body(p_ref, x_ref, dinv_ref, xs_ref, *, NC):
    deg = p_ref[0]
    for c in range(1, NC):
        deg = deg + p_ref[c]
    dinv = lax.rsqrt(deg + 1.0)
    dinv_ref[...] = dinv
    xs_ref[...] = x_ref[...] * dinv


def _mlp_body(p_ref, xs_ref, dinv_ref, W1_ref, b1_ref, W2_ref, ts_ref, *, NC):
    agg = p_ref[0]
    for c in range(1, NC):
        agg = agg + p_ref[c]
    ax = (agg + xs_ref[...]) * dinv_ref[...]
    h = jnp.dot(ax, W1_ref[...],
                preferred_element_type=jnp.float32) + b1_ref[...]
    h = jnp.maximum(h, 0.0)
    t = jnp.dot(h, W2_ref[...],
                preferred_element_type=jnp.float32)
    ts_ref[...] = t * dinv_ref[...]


def _out_body(p_ref, ts_ref, dinv_ref, b2_ref, o_ref, *, NC, D_OUT):
    agg = p_ref[0]
    for c in range(1, NC):
        agg = agg + p_ref[c]
    o = (agg + ts_ref[...]) * dinv_ref[...] + b2_ref[...]
    o = o[:, :D_OUT]
    m = jnp.max(o, axis=1, keepdims=True)
    e = jnp.exp(o - m)
    ssum = jnp.sum(e, axis=1, keepdims=True)
    o_ref[...] = o - m - jnp.log(ssum)


def kernel(x, edge_index, W1, b1, W2, b2):
    N, D_IN = x.shape
    E = edge_index.shape[1]
    D_HID = W1.shape[1]
    D_OUT = W2.shape[1]
    D_OUT_P = 48

    src = edge_index[0].astype(jnp.int32)
    dst = edge_index[1].astype(jnp.int32)
    W2p = jnp.pad(W2, ((0, 0), (0, D_OUT_P - D_OUT)))
    b1r = b1.reshape(1, D_HID)
    b2p = jnp.pad(b2, (0, D_OUT_P - D_OUT)).reshape(1, D_OUT_P)

    info = plsc.get_sparse_core_info()
    NC, NS = info.num_cores, info.num_subcores
    NW = NC * NS
    K = 80
    epw = E // NW
    N_pad = ((N + 16 * NS - 1) // (16 * NS)) * (16 * NS)

    src3 = src.reshape(NW, epw // K, K)
    dst3 = dst.reshape(NW, epw // K, K)

    deg_raw = _make_deg(N_pad, E, NC, NS, K)(dst3)
    deg_p = deg_raw[:, :N].reshape(NC, N, 1)

    BR = 5000
    dinv, xs = pl.pallas_call(
        functools.partial(_scale_body, NC=NC),
        grid=(N // BR,),
        in_specs=[
            pl.BlockSpec((NC, BR, 1), lambda i: (0, i, 0)),
            pl.BlockSpec((BR, D_IN), lambda i: (i, 0)),
        ],
        out_specs=[
            pl.BlockSpec((BR, 1), lambda i: (i, 0)),
            pl.BlockSpec((BR, D_IN), lambda i: (i, 0)),
        ],
        out_shape=[
            jax.ShapeDtypeStruct((N, 1), jnp.float32),
            jax.ShapeDtypeStruct((N, D_IN), jnp.float32),
        ],
    )(deg_p, x)

    p1 = _make_prop(N, D_IN, E, NC, NS, K, 3)(xs, src3, dst3)

    ts = pl.pallas_call(
        functools.partial(_mlp_body, NC=NC),
        grid=(N // BR,),
        in_specs=[
            pl.BlockSpec((NC, BR, D_IN), lambda i: (0, i, 0)),
            pl.BlockSpec((BR, D_IN), lambda i: (i, 0)),
            pl.BlockSpec((BR, 1), lambda i: (i, 0)),
            pl.BlockSpec((D_IN, D_HID), lambda i: (0, 0)),
            pl.BlockSpec((1, D_HID), lambda i: (0, 0)),
            pl.BlockSpec((D_HID, D_OUT_P), lambda i: (0, 0)),
        ],
        out_specs=pl.BlockSpec((BR, D_OUT_P), lambda i: (i, 0)),
        out_shape=jax.ShapeDtypeStruct((N, D_OUT_P), jnp.float32),
    )(p1, xs, dinv, W1, b1r, W2p)

    p2 = _make_prop(N, D_OUT_P, E, NC, NS, K, 6)(ts, src3, dst3)

    out = pl.pallas_call(
        functools.partial(_out_body, NC=NC, D_OUT=D_OUT),
        grid=(N // BR,),
        in_specs=[
            pl.BlockSpec((NC, BR, D_OUT_P), lambda i: (0, i, 0)),
            pl.BlockSpec((BR, D_OUT_P), lambda i: (i, 0)),
            pl.BlockSpec((BR, 1), lambda i: (i, 0)),
            pl.BlockSpec((1, D_OUT_P), lambda i: (0, 0)),
        ],
        out_specs=pl.BlockSpec((BR, D_OUT), lambda i: (i, 0)),
        out_shape=jax.ShapeDtypeStruct((N, D_OUT), jnp.float32),
    )(p2, ts, dinv, b2p)
    return out
